# Initial kernel scaffold; baseline (speedup 1.0000x reference)
import jax
import jax.numpy as jnp
from jax.experimental import pallas as pl


def kernel(r_idx, e1_idx, e2_idx, e3_idx, e4_idx, e5_idx, e6_idx, E, R):
    raise NotImplementedError("write your pallas kernel here")


# speedup vs baseline: 3.1962x; 3.1962x over previous
import functools

import jax
import jax.numpy as jnp
from jax import lax
from jax.experimental import pallas as pl
from jax.experimental.pallas import tpu as pltpu
from jax.experimental.pallas import tpu_sc as plsc

EMB = 128
ARITY = 6
SHIFTS = tuple(int(k * EMB / ARITY) for k in range(ARITY))
LANES = 16
NVREG = EMB // LANES


@functools.lru_cache(maxsize=None)
def _make_sc_kernel(batch):
    info = plsc.get_sparse_core_info()
    nc, ns = info.num_cores, info.num_subcores
    nw = nc * ns
    bpw = batch // nw
    C = 64
    nchunk = bpw // C
    SUMW = 17

    mesh = plsc.VectorSubcoreMesh(core_axis_name="c", subcore_axis_name="s")

    @functools.partial(
        pl.kernel,
        out_type=jax.ShapeDtypeStruct((batch,), jnp.float32),
        mesh=mesh,
        compiler_params=pltpu.CompilerParams(needs_layout_passes=False),
        scratch_types=[
            pltpu.VMEM((7, nchunk, C), jnp.int32),
            pltpu.VMEM((2, 7, C, EMB), jnp.float32),
            pltpu.VMEM((C * SUMW,), jnp.float32),
            pltpu.VMEM((bpw,), jnp.float32),
            pltpu.SemaphoreType.DMA,
            pltpu.SemaphoreType.DMA,
        ],
    )
    def sc_kernel(r_h, e1_h, e2_h, e3_h, e4_h, e5_h, e6_h, E_h, R_h,
                  out_h, idx_v, rows_v, sums_v, out_v, sem0, sem1):
        wid = lax.axis_index("s") * nc + lax.axis_index("c")
        base = wid * bpw
        idx_srcs = (r_h, e1_h, e2_h, e3_h, e4_h, e5_h, e6_h)
        for k in range(7):
            for c in range(nchunk):
                pltpu.sync_copy(idx_srcs[k].at[pl.ds(base + c * C, C)],
                                idx_v.at[k, c])

        sems = (sem0, sem1)

        def issue(c):
            p = c & 1
            cps = []
            for k in range(7):
                tab = R_h if k == 0 else E_h
                cps.append(pltpu.async_copy(tab.at[idx_v.at[k, c]],
                                            rows_v.at[p, k], sems[p]))
            return cps

        lane = lax.iota(jnp.int32, 16)
        cols = []
        for k in range(7):
            sh = 0 if k == 0 else SHIFTS[k - 1]
            cols.append([(lane + (LANES * i + sh)) & (EMB - 1)
                         for i in range(NVREG)])
        tsum_base = lane * SUMW

        def compute(c):
            p = c & 1

            p_splat = jnp.full((16,), p, jnp.int32)
            k_splats = [jnp.full((16,), k, jnp.int32) for k in range(7)]

            def row_body(r, carry):
                refs = [rows_v.at[p, k, r] for k in range(7)]
                r_splat = jnp.broadcast_to(r, (16,))
                prod = [refs[0][pl.ds(LANES * i, LANES)]
                        * refs[1][pl.ds(LANES * i, LANES)]
                        for i in range(NVREG)]
                for k in range(2, 7):
                    sh = SHIFTS[k - 1]
                    for i in range(NVREG):
                        lo = LANES * i + sh
                        if (lo % EMB) + LANES <= EMB:
                            v = refs[k][pl.ds(lo % EMB, LANES)]
                        else:
                            v = plsc.load_gather(
                                rows_v,
                                [p_splat, k_splats[k], r_splat, cols[k][i]])
                        prod[i] = prod[i] * v
                s01 = prod[0] + prod[1]
                s23 = prod[2] + prod[3]
                s45 = prod[4] + prod[5]
                s67 = prod[6] + prod[7]
                sums_v[pl.ds(r * SUMW, LANES)] = (s01 + s23) + (s45 + s67)
                return carry

            lax.fori_loop(0, C, row_body, 0)

            for g in range(C // LANES):
                acc = None
                for j in range(LANES):
                    col = plsc.load_gather(
                        sums_v, [tsum_base + (g * LANES * SUMW + j)])
                    acc = col if acc is None else acc + col
                out_v[pl.ds(c * C + g * LANES, LANES)] = acc

        pending = issue(0)
        for c in range(nchunk):
            for cp in pending:
                cp.wait()
            if c + 1 < nchunk:
                nxt = issue(c + 1)
            compute(c)
            if c + 1 < nchunk:
                pending = nxt

        pltpu.sync_copy(out_v, out_h.at[pl.ds(base, bpw)])

    return sc_kernel


def kernel(r_idx, e1_idx, e2_idx, e3_idx, e4_idx, e5_idx, e6_idx, E, R):
    batch = r_idx.shape[0]
    f = _make_sc_kernel(batch)
    idxs = [jnp.asarray(a, jnp.int32)
            for a in (r_idx, e1_idx, e2_idx, e3_idx, e4_idx, e5_idx, e6_idx)]
    return f(*idxs, E, R)


# speedup vs baseline: 4.0797x; 1.2764x over previous
import functools

import jax
import jax.numpy as jnp
from jax import lax
from jax.experimental import pallas as pl
from jax.experimental.pallas import tpu as pltpu
from jax.experimental.pallas import tpu_sc as plsc

EMB = 128
ARITY = 6
SHIFTS = tuple(int(k * EMB / ARITY) for k in range(ARITY))
LANES = 16
NVREG = EMB // LANES


@functools.lru_cache(maxsize=None)
def _make_sc_kernel(batch):
    info = plsc.get_sparse_core_info()
    nc, ns = info.num_cores, info.num_subcores
    nw = nc * ns
    bpw = batch // nw
    C = 64
    nchunk = bpw // C
    SUMW = 17

    mesh = plsc.VectorSubcoreMesh(core_axis_name="c", subcore_axis_name="s")

    @functools.partial(
        pl.kernel,
        out_type=jax.ShapeDtypeStruct((batch,), jnp.float32),
        mesh=mesh,
        compiler_params=pltpu.CompilerParams(needs_layout_passes=False),
        scratch_types=[
            pltpu.VMEM((7, nchunk, C), jnp.int32),
            pltpu.VMEM((2, 7, C, EMB), jnp.float32),
            pltpu.VMEM((C * SUMW,), jnp.float32),
            pltpu.VMEM((bpw,), jnp.float32),
            pltpu.SemaphoreType.DMA,
            pltpu.SemaphoreType.DMA,
        ],
    )
    def sc_kernel(r_h, e1_h, e2_h, e3_h, e4_h, e5_h, e6_h, E_h, R_h,
                  out_h, idx_v, rows_v, sums_v, out_v, sem0, sem1):
        wid = lax.axis_index("s") * nc + lax.axis_index("c")
        base = wid * bpw
        idx_srcs = (r_h, e1_h, e2_h, e3_h, e4_h, e5_h, e6_h)
        stage = [pltpu.async_copy(idx_srcs[k].at[wid], idx_v.at[k], sem0)
                 for k in range(7)]
        for cp in stage:
            cp.wait()

        sems = (sem0, sem1)

        def issue(c):
            p = c & 1
            cps = []
            for k in range(7):
                tab = R_h if k == 0 else E_h
                cps.append(pltpu.async_copy(tab.at[idx_v.at[k, c]],
                                            rows_v.at[p, k], sems[p]))
            return cps

        lane = lax.iota(jnp.int32, 16)
        cols = []
        for k in range(7):
            sh = 0 if k == 0 else SHIFTS[k - 1]
            cols.append([(lane + (LANES * i + sh)) & (EMB - 1)
                         for i in range(NVREG)])
        tsum_base = lane * SUMW

        def compute(c):
            p = c & 1

            p_splat = jnp.full((16,), p, jnp.int32)
            k_splats = [jnp.full((16,), k, jnp.int32) for k in range(7)]

            def one_row(r, r_splat):
                refs = [rows_v.at[p, k, r] for k in range(7)]
                prod = [refs[0][pl.ds(LANES * i, LANES)]
                        * refs[1][pl.ds(LANES * i, LANES)]
                        for i in range(NVREG)]
                for k in range(2, 7):
                    sh = SHIFTS[k - 1]
                    for i in range(NVREG):
                        lo = LANES * i + sh
                        if (lo % EMB) + LANES <= EMB:
                            v = refs[k][pl.ds(lo % EMB, LANES)]
                        else:
                            v = plsc.load_gather(
                                rows_v,
                                [p_splat, k_splats[k], r_splat, cols[k][i]])
                        prod[i] = prod[i] * v
                s01 = prod[0] + prod[1]
                s23 = prod[2] + prod[3]
                s45 = prod[4] + prod[5]
                s67 = prod[6] + prod[7]
                sums_v[pl.ds(r * SUMW, LANES)] = (s01 + s23) + (s45 + s67)

            def row_body(it, carry):
                r0 = it * 2
                one_row(r0, jnp.broadcast_to(r0, (16,)))
                one_row(r0 + 1, jnp.broadcast_to(r0 + 1, (16,)))
                return carry

            lax.fori_loop(0, C // 2, row_body, 0)

            for g in range(C // LANES):
                acc = None
                for j in range(LANES):
                    col = plsc.load_gather(
                        sums_v, [tsum_base + (g * LANES * SUMW + j)])
                    acc = col if acc is None else acc + col
                out_v[pl.ds(c * C + g * LANES, LANES)] = acc

        pending = issue(0)
        for c in range(nchunk):
            for cp in pending:
                cp.wait()
            if c + 1 < nchunk:
                nxt = issue(c + 1)
            compute(c)
            if c + 1 < nchunk:
                pending = nxt

        pltpu.sync_copy(out_v, out_h.at[pl.ds(base, bpw)])

    return sc_kernel


def kernel(r_idx, e1_idx, e2_idx, e3_idx, e4_idx, e5_idx, e6_idx, E, R):
    batch = r_idx.shape[0]
    info = plsc.get_sparse_core_info()
    nw = info.num_cores * info.num_subcores
    C = 64
    nchunk = batch // (nw * C)
    f = _make_sc_kernel(batch)
    idxs = [jnp.asarray(a, jnp.int32).reshape(nw, nchunk, C)
            for a in (r_idx, e1_idx, e2_idx, e3_idx, e4_idx, e5_idx, e6_idx)]
    return f(*idxs, E, R)


# speedup vs baseline: 4.4823x; 1.0987x over previous
import functools

import jax
import jax.numpy as jnp
from jax import lax
from jax.experimental import pallas as pl
from jax.experimental.pallas import tpu as pltpu
from jax.experimental.pallas import tpu_sc as plsc

EMB = 128
ARITY = 6
SHIFTS = tuple(int(k * EMB / ARITY) for k in range(ARITY))
LANES = 16
NVREG = EMB // LANES


@functools.lru_cache(maxsize=None)
def _make_sc_kernel(batch):
    info = plsc.get_sparse_core_info()
    nc, ns = info.num_cores, info.num_subcores
    nw = nc * ns
    bpw = batch // nw
    C = 64
    nchunk = bpw // C
    SUMW = 17

    mesh = plsc.VectorSubcoreMesh(core_axis_name="c", subcore_axis_name="s")

    @functools.partial(
        pl.kernel,
        out_type=jax.ShapeDtypeStruct((batch,), jnp.float32),
        mesh=mesh,
        compiler_params=pltpu.CompilerParams(needs_layout_passes=False),
        scratch_types=[
            pltpu.VMEM((7, nchunk, C), jnp.int32),
            pltpu.VMEM((2, 7, C, EMB), jnp.float32),
            pltpu.VMEM((C * SUMW,), jnp.float32),
            pltpu.VMEM((bpw,), jnp.float32),
            pltpu.SemaphoreType.DMA,
            pltpu.SemaphoreType.DMA,
        ],
    )
    def sc_kernel(r_h, e1_h, e2_h, e3_h, e4_h, e5_h, e6_h, E_h, R_h,
                  out_h, idx_v, rows_v, sums_v, out_v, sem0, sem1):
        wid = lax.axis_index("s") * nc + lax.axis_index("c")
        base = wid * bpw
        idx_srcs = (r_h, e1_h, e2_h, e3_h, e4_h, e5_h, e6_h)
        stage = []
        for k in range(7):
            for c in range(nchunk):
                stage.append(pltpu.async_copy(
                    idx_srcs[k].at[pl.ds(base + c * C, C)],
                    idx_v.at[k, c], sem0))
        for cp in stage:
            cp.wait()

        sems = (sem0, sem1)

        def issue(c):
            p = c & 1
            cps = []
            for k in range(7):
                tab = R_h if k == 0 else E_h
                cps.append(pltpu.async_copy(tab.at[idx_v.at[k, c]],
                                            rows_v.at[p, k], sems[p]))
            return cps

        lane = lax.iota(jnp.int32, 16)
        cols = []
        for k in range(7):
            sh = 0 if k == 0 else SHIFTS[k - 1]
            cols.append([(lane + (LANES * i + sh)) & (EMB - 1)
                         for i in range(NVREG)])
        tsum_base = lane * SUMW

        def compute(c):
            p = c & 1

            p_splat = jnp.full((16,), p, jnp.int32)
            k_splats = [jnp.full((16,), k, jnp.int32) for k in range(7)]

            def one_row(r, r_splat):
                refs = [rows_v.at[p, k, r] for k in range(7)]
                prod = [refs[0][pl.ds(LANES * i, LANES)]
                        * refs[1][pl.ds(LANES * i, LANES)]
                        for i in range(NVREG)]
                for k in range(2, 7):
                    sh = SHIFTS[k - 1]
                    for i in range(NVREG):
                        lo = LANES * i + sh
                        if (lo % EMB) + LANES <= EMB:
                            v = refs[k][pl.ds(lo % EMB, LANES)]
                        else:
                            v = plsc.load_gather(
                                rows_v,
                                [p_splat, k_splats[k], r_splat, cols[k][i]])
                        prod[i] = prod[i] * v
                s01 = prod[0] + prod[1]
                s23 = prod[2] + prod[3]
                s45 = prod[4] + prod[5]
                s67 = prod[6] + prod[7]
                sums_v[pl.ds(r * SUMW, LANES)] = (s01 + s23) + (s45 + s67)

            def row_body(it, carry):
                r0 = it * 2
                one_row(r0, jnp.broadcast_to(r0, (16,)))
                one_row(r0 + 1, jnp.broadcast_to(r0 + 1, (16,)))
                return carry

            lax.fori_loop(0, C // 2, row_body, 0)

            for g in range(C // LANES):
                acc = None
                for j in range(LANES):
                    col = plsc.load_gather(
                        sums_v, [tsum_base + (g * LANES * SUMW + j)])
                    acc = col if acc is None else acc + col
                out_v[pl.ds(c * C + g * LANES, LANES)] = acc

        pending = issue(0)
        for c in range(nchunk):
            for cp in pending:
                cp.wait()
            if c + 1 < nchunk:
                nxt = issue(c + 1)
            compute(c)
            if c + 1 < nchunk:
                pending = nxt

        pltpu.sync_copy(out_v, out_h.at[pl.ds(base, bpw)])

    return sc_kernel


def kernel(r_idx, e1_idx, e2_idx, e3_idx, e4_idx, e5_idx, e6_idx, E, R):
    batch = r_idx.shape[0]
    f = _make_sc_kernel(batch)
    idxs = [jnp.asarray(a, jnp.int32)
            for a in (r_idx, e1_idx, e2_idx, e3_idx, e4_idx, e5_idx, e6_idx)]
    return f(*idxs, E, R)
